# baseline (speedup 1.0000x reference)
import functools

import jax
import jax.numpy as jnp
from jax import lax
from jax.experimental import pallas as pl
from jax.experimental.pallas import tpu as pltpu
from jax.experimental.pallas import tpu_sc as plsc


def _build_gather(B, R, D, dtype):
    info = plsc.get_sparse_core_info()
    NC, NS = info.num_cores, info.num_subcores
    NW = NC * NS
    assert B % NW == 0
    bpw = B // NW
    H = bpw // 2

    mesh = plsc.VectorSubcoreMesh(core_axis_name="c", subcore_axis_name="s")

    @functools.partial(
        pl.kernel,
        mesh=mesh,
        out_type=jax.ShapeDtypeStruct((B, R, D), dtype),
        compiler_params=pltpu.CompilerParams(use_tc_tiling_on_sc=False),
        scratch_types=[
            pltpu.VMEM((bpw, R), jnp.int32),
            pltpu.VMEM((H, R, 2 * D), dtype),
            pltpu.SemaphoreType.DMA,
            pltpu.SemaphoreType.DMA,
        ],
    )
    def k(tab_hbm, lab_hbm, tags_hbm, idx_v, t_v, sem_g, sem_o):
        wid = lax.axis_index("s") * NC + lax.axis_index("c")
        base = wid * bpw
        pltpu.sync_copy(lab_hbm.at[pl.ds(base, bpw)], idx_v)
        for h in range(2):
            b0 = base + h * H

            def fire(j, carry):
                pltpu.async_copy(
                    tab_hbm.at[idx_v.at[h * H + j]], t_v.at[j], sem_g)
                return carry

            lax.fori_loop(0, H, fire, 0)
            pltpu.make_async_copy(
                tags_hbm.at[pl.ds(0, H)],
                t_v.at[:, :, pl.ds(0, D)], sem_g).wait()
            pltpu.make_async_copy(
                tags_hbm.at[pl.ds(0, H)],
                t_v.at[:, :, pl.ds(D, D)], sem_g).wait()
            pltpu.sync_copy(
                t_v.at[:, :, pl.ds(0, D)], tags_hbm.at[pl.ds(b0, H)])

    return k


def _repack_rowmajor(tag_table, DW):
    V, D = tag_table.shape
    dt = tag_table.dtype
    pad_eye = jnp.concatenate(
        [jnp.eye(D, dtype=dt), jnp.zeros((D, DW - D), dtype=dt)], axis=1)
    return tag_table @ pad_eye


def kernel(global_feat, region_feats, tag_table, labels):
    B, R, D = region_feats.shape
    tab = _repack_rowmajor(tag_table, 2 * D)
    k = _build_gather(B, R, D, region_feats.dtype)
    tags = k(tab, labels)
    return jnp.concatenate([global_feat, region_feats, tags], axis=1)
